# baseline (speedup 1.0000x reference)
import jax
import jax.numpy as jnp
from jax import lax
from jax.experimental import pallas as pl

NUM_CLASSES = 10
NUM_YAW = 2
BOX_DOF = 7
C_IN = 384
B, NY, NX = 4, 200, 176
HW = NY * NX
TILE = 1408
C_CLS = NUM_CLASSES * NUM_YAW
C_REG = C_CLS * BOX_DOF


def _head_kernel(x_ref, wc_ref, bc_ref, wr_ref, br_ref, cls_ref, reg_ref):
    x = x_ref[0]
    cls_ref[0] = (
        jnp.dot(wc_ref[...], x, preferred_element_type=jnp.float32) + bc_ref[...]
    )
    reg = (
        lax.dot_general(
            x, wr_ref[...],
            dimension_numbers=(((0,), (1,)), ((), ())),
            preferred_element_type=jnp.float32,
        )
        + br_ref[...]
    )
    for cy in range(C_CLS):
        reg_ref[0, cy] = reg[:, cy * BOX_DOF:(cy + 1) * BOX_DOF]


def kernel(feature_map, W_cls, b_cls, W_reg, b_reg):
    x = feature_map.reshape(B, C_IN, HW)
    bc = b_cls.reshape(C_CLS, 1)
    perm = jnp.asarray(
        [c * 14 + d * 2 + y
         for c in range(NUM_CLASSES)
         for y in range(NUM_YAW)
         for d in range(BOX_DOF)],
        dtype=jnp.int32,
    )
    wr = W_reg[perm]
    br = b_reg[perm].reshape(1, C_REG)

    nt = HW // TILE
    cls_out, reg_out = pl.pallas_call(
        _head_kernel,
        grid=(B, nt),
        in_specs=[
            pl.BlockSpec((1, C_IN, TILE), lambda b, t: (b, 0, t)),
            pl.BlockSpec((C_CLS, C_IN), lambda b, t: (0, 0)),
            pl.BlockSpec((C_CLS, 1), lambda b, t: (0, 0)),
            pl.BlockSpec((C_REG, C_IN), lambda b, t: (0, 0)),
            pl.BlockSpec((1, C_REG), lambda b, t: (0, 0)),
        ],
        out_specs=[
            pl.BlockSpec((1, C_CLS, TILE), lambda b, t: (b, 0, t)),
            pl.BlockSpec((1, C_CLS, TILE, BOX_DOF), lambda b, t: (b, 0, t, 0)),
        ],
        out_shape=[
            jax.ShapeDtypeStruct((B, C_CLS, HW), jnp.float32),
            jax.ShapeDtypeStruct((B, C_CLS, HW, BOX_DOF), jnp.float32),
        ],
    )(x, W_cls, bc, wr, br)

    cls_map = cls_out.reshape(B, NUM_CLASSES, NUM_YAW, NY, NX)
    reg_map = reg_out.reshape(B, NUM_CLASSES, NUM_YAW, NY, NX, BOX_DOF)
    return (cls_map, reg_map)


# speedup vs baseline: 3.9870x; 3.9870x over previous
import jax
import jax.numpy as jnp
from jax import lax
from jax.experimental import pallas as pl

NUM_CLASSES = 10
NUM_YAW = 2
BOX_DOF = 7
C_IN = 384
B, NY, NX = 4, 200, 176
C_CLS = NUM_CLASSES * NUM_YAW
C_REG = C_CLS * BOX_DOF
PAD = 4
C_ALL = C_CLS + PAD + C_REG
TX = 8


def _head_kernel(x_ref, w_ref, b_ref, cls_ref, reg_ref):
    x = x_ref[0].reshape(NY * TX, C_IN)
    r = (
        jnp.dot(x, w_ref[...], preferred_element_type=jnp.float32)
        + b_ref[...]
    )
    r3 = r.reshape(NY, TX, C_ALL)
    v = jnp.transpose(r3, (2, 1, 0))
    cls_ref[0] = v[0:C_CLS].reshape(NUM_CLASSES, NUM_YAW, TX, NY)
    reg_ref[0] = v[C_CLS + PAD:].reshape(
        NUM_CLASSES, NUM_YAW, BOX_DOF, TX, NY
    )


def kernel(feature_map, W_cls, b_cls, W_reg, b_reg):
    xt = jnp.transpose(feature_map, (0, 2, 3, 1))

    perm = jnp.asarray(
        [c * 14 + d * 2 + y
         for c in range(NUM_CLASSES)
         for y in range(NUM_YAW)
         for d in range(BOX_DOF)],
        dtype=jnp.int32,
    )
    w_all = jnp.concatenate(
        [W_cls, jnp.zeros((PAD, C_IN), jnp.float32), W_reg[perm]], axis=0
    ).T
    b_all = jnp.concatenate(
        [b_cls, jnp.zeros((PAD,), jnp.float32), b_reg[perm]], axis=0
    ).reshape(1, C_ALL)

    nt = NX // TX
    cls_t, reg_t = pl.pallas_call(
        _head_kernel,
        grid=(B, nt),
        in_specs=[
            pl.BlockSpec((1, NY, TX, C_IN), lambda b, t: (b, 0, t, 0)),
            pl.BlockSpec((C_IN, C_ALL), lambda b, t: (0, 0)),
            pl.BlockSpec((1, C_ALL), lambda b, t: (0, 0)),
        ],
        out_specs=[
            pl.BlockSpec(
                (1, NUM_CLASSES, NUM_YAW, TX, NY), lambda b, t: (b, 0, 0, t, 0)
            ),
            pl.BlockSpec(
                (1, NUM_CLASSES, NUM_YAW, BOX_DOF, TX, NY),
                lambda b, t: (b, 0, 0, 0, t, 0),
            ),
        ],
        out_shape=[
            jax.ShapeDtypeStruct((B, NUM_CLASSES, NUM_YAW, NX, NY), jnp.float32),
            jax.ShapeDtypeStruct(
                (B, NUM_CLASSES, NUM_YAW, BOX_DOF, NX, NY), jnp.float32
            ),
        ],
    )(xt, w_all, b_all)

    cls_map = jnp.transpose(cls_t, (0, 1, 2, 4, 3))
    reg_map = jnp.transpose(reg_t, (0, 1, 2, 5, 4, 3))
    return (cls_map, reg_map)
